# Initial kernel scaffold; baseline (speedup 1.0000x reference)
import jax
import jax.numpy as jnp
from jax.experimental import pallas as pl


def kernel(table, index, targets):
    raise NotImplementedError("write your pallas kernel here")


# speedup vs baseline: 1.7123x; 1.7123x over previous
import functools

import jax
import jax.numpy as jnp
from jax.experimental import pallas as pl

_VOCAB = 1000
_BLK = 256


def _fused_kernel(idx_ref, tgt_ref, table_ref, out_ref, loss_ref):
    idx = idx_ref[0, 0, :]
    tgt = tgt_ref[0, 0, :]
    tab = table_ref[...]
    iota = jax.lax.broadcasted_iota(jnp.int32, (_BLK, _VOCAB), 1)
    onehot = (idx[:, None] == iota).astype(jnp.float32)
    logits = jnp.dot(onehot, tab, preferred_element_type=jnp.float32)
    out_ref[...] = logits
    m = jnp.max(logits, axis=1)
    lse = m + jnp.log(jnp.sum(jnp.exp(logits - m[:, None]), axis=1))
    picked = jnp.sum(jnp.where(tgt[:, None] == iota, logits, 0.0), axis=1)
    part = jnp.sum(lse - picked).reshape(1, 1)

    @pl.when(pl.program_id(0) == 0)
    def _init():
        loss_ref[...] = jnp.zeros((1, 1), jnp.float32)

    loss_ref[...] += part


@functools.partial(jax.jit, static_argnames=())
def kernel(table, index, targets):
    b, t = index.shape
    n = b * t
    nblk = n // _BLK
    idx = index.reshape(nblk, 1, _BLK).astype(jnp.int32)
    tgt = targets.reshape(nblk, 1, _BLK).astype(jnp.int32)

    logits_flat, loss_sum = pl.pallas_call(
        _fused_kernel,
        grid=(nblk,),
        in_specs=[
            pl.BlockSpec((1, 1, _BLK), lambda i: (i, 0, 0)),
            pl.BlockSpec((1, 1, _BLK), lambda i: (i, 0, 0)),
            pl.BlockSpec((_VOCAB, _VOCAB), lambda i: (0, 0)),
        ],
        out_specs=[
            pl.BlockSpec((_BLK, _VOCAB), lambda i: (i, 0)),
            pl.BlockSpec((1, 1), lambda i: (0, 0)),
        ],
        out_shape=[
            jax.ShapeDtypeStruct((n, _VOCAB), jnp.float32),
            jax.ShapeDtypeStruct((1, 1), jnp.float32),
        ],
    )(idx, tgt, table)

    logits = logits_flat.reshape(b, t, _VOCAB)
    loss = loss_sum[0, 0] / n
    return (logits, loss)
